# baseline (speedup 1.0000x reference)
import jax
import jax.numpy as jnp
from jax.experimental import pallas as pl

B = 16
T = 50
P = 8
H = 64
W = 64
N = B * T
G = 40


def _body(xr, yr, resr, orgr, out_ref):
    res = resr[...]
    org = orgr[...]
    coli = (xr[...] / res[:, 0:1] + org[:, 1:2]).astype(jnp.int32)
    rowi = (yr[...] / res[:, 1:2] + org[:, 0:1]).astype(jnp.int32)
    inb = (coli >= 0) & (coli < W) & (rowi >= 0) & (rowi < H)
    tgt_r = jnp.where(inb, rowi, -1)
    hio = jax.lax.broadcasted_iota(jnp.int32, (G, H, P, W), 1)
    wio = jax.lax.broadcasted_iota(jnp.int32, (G, H, P, W), 3)
    hit = (hio == tgt_r[:, None, :, None]) & (wio == coli[:, None, :, None])
    out_ref[...] = hit.astype(jnp.float32)


def kernel(x, resolution, origin):
    pts = x.reshape(N, P, 2)
    xc = pts[:, :, 0]
    yc = pts[:, :, 1]
    res = resolution.reshape(N, 2)
    org = origin.reshape(N, 2)

    out = pl.pallas_call(
        _body,
        grid=(N // G,),
        in_specs=[
            pl.BlockSpec((G, P), lambda i: (i, 0)),
            pl.BlockSpec((G, P), lambda i: (i, 0)),
            pl.BlockSpec((G, 2), lambda i: (i, 0)),
            pl.BlockSpec((G, 2), lambda i: (i, 0)),
        ],
        out_specs=pl.BlockSpec((G, H, P, W), lambda i: (i, 0, 0, 0)),
        out_shape=jax.ShapeDtypeStruct((N, H, P, W), jnp.float32),
    )(xc, yc, res, org)
    out5 = out.reshape(B, T, H, P, W)
    return jnp.transpose(out5, (0, 1, 2, 4, 3))
